# baseline (speedup 1.0000x reference)
import functools

import jax
import jax.numpy as jnp
from jax import lax
from jax.experimental import pallas as pl
from jax.experimental.pallas import tpu as pltpu
from jax.experimental.pallas import tpu_sc as plsc

N = 10000
E = 320000
D_IN = 128
H = 256

M_PAD = 10240
BLK_M = 2048
BLK_E = 8000
HALF = M_PAD // 2
TRASH = HALF
AGG_ROWS = HALF + 8
NSUB = 16
E_PER_TILE = E // NSUB
BE = 32
NB = E_PER_TILE // BE
ROWS_PER_TILE = HALF // NSUB


def _mm_in(x, W_in, b_in, Wm0):
    def body(x_ref, w_ref, b_ref, wm_ref, h_ref, hw_ref):
        h = jnp.maximum(
            jnp.dot(x_ref[...], w_ref[...], preferred_element_type=jnp.float32)
            + b_ref[...], 0.0)
        h_ref[...] = h
        hw_ref[...] = jnp.dot(h, wm_ref[...], preferred_element_type=jnp.float32)

    return pl.pallas_call(
        body,
        grid=(M_PAD // BLK_M,),
        in_specs=[
            pl.BlockSpec((BLK_M, D_IN), lambda i: (i, 0)),
            pl.BlockSpec((D_IN, H), lambda i: (0, 0)),
            pl.BlockSpec((1, H), lambda i: (0, 0)),
            pl.BlockSpec((H, H), lambda i: (0, 0)),
        ],
        out_specs=[
            pl.BlockSpec((BLK_M, H), lambda i: (i, 0)),
            pl.BlockSpec((BLK_M, H), lambda i: (i, 0)),
        ],
        out_shape=[
            jax.ShapeDtypeStruct((M_PAD, H), jnp.float32),
            jax.ShapeDtypeStruct((M_PAD, H), jnp.float32),
        ],
    )(x, W_in, b_in, Wm0)


def _mm_ew(edge_attr, We):
    def body(a_ref, w_ref, o_ref):
        o_ref[...] = jnp.dot(a_ref[...], w_ref[...],
                             preferred_element_type=jnp.float32)

    return pl.pallas_call(
        body,
        grid=(E // BLK_E,),
        in_specs=[
            pl.BlockSpec((BLK_E, 16), lambda i: (i, 0)),
            pl.BlockSpec((16, H), lambda i: (0, 0)),
        ],
        out_specs=pl.BlockSpec((BLK_E, H), lambda i: (i, 0)),
        out_shape=jax.ShapeDtypeStruct((E, H), jnp.float32),
    )(edge_attr, We)


def _mm_update(h, agg, Ws, Wa, b, Wm_next):
    def body(h_ref, g_ref, ws_ref, wa_ref, b_ref, wm_ref, hn_ref, hw_ref):
        hn = jnp.maximum(
            jnp.dot(h_ref[...], ws_ref[...], preferred_element_type=jnp.float32)
            + jnp.dot(g_ref[...], wa_ref[...], preferred_element_type=jnp.float32)
            + b_ref[...], 0.0)
        hn_ref[...] = hn
        hw_ref[...] = jnp.dot(hn, wm_ref[...], preferred_element_type=jnp.float32)

    return pl.pallas_call(
        body,
        grid=(M_PAD // BLK_M,),
        in_specs=[
            pl.BlockSpec((BLK_M, H), lambda i: (i, 0)),
            pl.BlockSpec((BLK_M, H), lambda i: (i, 0)),
            pl.BlockSpec((H, H), lambda i: (0, 0)),
            pl.BlockSpec((H, H), lambda i: (0, 0)),
            pl.BlockSpec((1, H), lambda i: (0, 0)),
            pl.BlockSpec((H, H), lambda i: (0, 0)),
        ],
        out_specs=[
            pl.BlockSpec((BLK_M, H), lambda i: (i, 0)),
            pl.BlockSpec((BLK_M, H), lambda i: (i, 0)),
        ],
        out_shape=[
            jax.ShapeDtypeStruct((M_PAD, H), jnp.float32),
            jax.ShapeDtypeStruct((M_PAD, H), jnp.float32),
        ],
    )(h, agg, Ws, Wa, b, Wm_next)


def _mm_final(h, agg, Ws, Wa, b, w_out_row, b_out):
    def body(h_ref, g_ref, ws_ref, wa_ref, b_ref, wo_ref, bo_ref, o_ref):
        hn = jnp.maximum(
            jnp.dot(h_ref[...], ws_ref[...], preferred_element_type=jnp.float32)
            + jnp.dot(g_ref[...], wa_ref[...], preferred_element_type=jnp.float32)
            + b_ref[...], 0.0)
        o_ref[...] = jnp.sum(hn * wo_ref[...], axis=1, keepdims=True) + bo_ref[...]

    return pl.pallas_call(
        body,
        grid=(M_PAD // BLK_M,),
        in_specs=[
            pl.BlockSpec((BLK_M, H), lambda i: (i, 0)),
            pl.BlockSpec((BLK_M, H), lambda i: (i, 0)),
            pl.BlockSpec((H, H), lambda i: (0, 0)),
            pl.BlockSpec((H, H), lambda i: (0, 0)),
            pl.BlockSpec((1, H), lambda i: (0, 0)),
            pl.BlockSpec((1, H), lambda i: (0, 0)),
            pl.BlockSpec((1, 1), lambda i: (0, 0)),
        ],
        out_specs=pl.BlockSpec((BLK_M, 1), lambda i: (i, 0)),
        out_shape=jax.ShapeDtypeStruct((M_PAD, 1), jnp.float32),
    )(h, agg, Ws, Wa, b, w_out_row, b_out)


NW = 32
RPT = M_PAD // NW
BD = 2000
NBLK = E // BD
SPILL = 1024
STAGE = 2 * SPILL + 16
CAP = 312 * SPILL + 2 * SPILL
PAD = 112


def _sc_prepass(src, dst):
    mesh = plsc.VectorSubcoreMesh(core_axis_name="c", subcore_axis_name="s")

    @functools.partial(
        pl.kernel,
        mesh=mesh,
        out_type=[
            jax.ShapeDtypeStruct((NW * CAP,), jnp.int32),
            jax.ShapeDtypeStruct((NW * CAP,), jnp.int32),
            jax.ShapeDtypeStruct((NW * CAP,), jnp.int32),
            jax.ShapeDtypeStruct((NW * 16,), jnp.int32),
        ],
        compiler_params=pltpu.CompilerParams(needs_layout_passes=False),
        scratch_types=[
            pltpu.VMEM((BD,), jnp.int32),
            pltpu.VMEM((BD,), jnp.int32),
            pltpu.VMEM((BD,), jnp.int32),
            pltpu.VMEM((BD,), jnp.int32),
            pltpu.VMEM((STAGE,), jnp.int32),
            pltpu.VMEM((STAGE,), jnp.int32),
            pltpu.VMEM((STAGE,), jnp.int32),
            pltpu.VMEM((16,), jnp.int32),
            pltpu.SemaphoreType.DMA,
            pltpu.SemaphoreType.DMA,
        ],
    )
    def k(src_hbm, dst_hbm, lsrc_hbm, leid_hbm, ldl_hbm, cnt_hbm,
          sblk0, sblk1, dblk0, dblk1, ssrc_v, seid_v, sdl_v, cnt_v,
          semb0, semb1):
        c = lax.axis_index("c")
        s = lax.axis_index("s")
        w = s * 2 + c
        lo = w * RPT
        base = w * CAP
        sblk = (sblk0, sblk1)
        dblk = (dblk0, dblk1)
        semb = (semb0, semb1)

        def blk_off(blk):
            return jnp.minimum(blk, NBLK - 1) * BD

        def fire_blk(h, blk):
            o = blk_off(blk)
            pltpu.async_copy(src_hbm.at[pl.ds(o, BD)], sblk[h], semb[h])
            pltpu.async_copy(dst_hbm.at[pl.ds(o, BD)], dblk[h], semb[h])

        def wait_blk(h, blk):
            o = blk_off(blk)
            pltpu.make_async_copy(src_hbm.at[pl.ds(o, BD)], sblk[h], semb[h]).wait()
            pltpu.make_async_copy(dst_hbm.at[pl.ds(o, BD)], dblk[h], semb[h]).wait()

        def spill_chunk(spilled):
            hsl = pl.ds(base + spilled * SPILL, SPILL)
            vsl = pl.ds(0, SPILL)
            pltpu.sync_copy(ssrc_v.at[vsl], lsrc_hbm.at[hsl])
            pltpu.sync_copy(seid_v.at[vsl], leid_hbm.at[hsl])
            pltpu.sync_copy(sdl_v.at[vsl], ldl_hbm.at[hsl])

        def scan_half(h, blk, carry):
            def chunk_body(i, carry):
                off, spilled = carry
                sl = pl.ds(i * 16, 16)
                dv = dblk[h][sl]
                sv = sblk[h][sl]
                dloc = dv - lo
                own = (dloc >= 0) & (dloc < RPT)
                eid = blk * BD + i * 16 + lax.iota(jnp.int32, 16)
                cnt = plsc.all_reduce_population_count(own)[0]

                def do_stage(off):
                    osl = pl.ds(off, 16)
                    plsc.store_compressed(ssrc_v.at[osl], sv, mask=own)
                    plsc.store_compressed(seid_v.at[osl], eid, mask=own)
                    plsc.store_compressed(sdl_v.at[osl], dloc, mask=own)
                    return off + cnt

                off = lax.cond(cnt > 0, do_stage, lambda off: off, off)

                def spill(args):
                    off, spilled = args
                    spill_chunk(spilled)
                    rsl = pl.ds(SPILL, 16)
                    rs, re, rd = ssrc_v[rsl], seid_v[rsl], sdl_v[rsl]
                    tsl = pl.ds(0, 16)
                    ssrc_v[tsl] = rs
                    seid_v[tsl] = re
                    sdl_v[tsl] = rd
                    return off - SPILL, spilled + 1

                return lax.cond(off >= SPILL, spill, lambda a: a, (off, spilled))
            return lax.fori_loop(0, BD // 16, chunk_body, carry)

        fire_blk(0, 0)

        def pair(p, carry):
            b0 = 2 * p
            wait_blk(0, b0)
            fire_blk(1, b0 + 1)
            carry = scan_half(0, b0, carry)
            wait_blk(1, b0 + 1)
            fire_blk(0, b0 + 2)
            carry = scan_half(1, b0 + 1, carry)
            return carry

        off, spilled = lax.fori_loop(0, NBLK // 2, pair,
                                     (jnp.int32(0), jnp.int32(0)))
        wait_blk(0, NBLK)

        pad_src = jnp.zeros((16,), jnp.int32)
        pad_dl = jnp.full((16,), RPT, jnp.int32)
        for t in range(PAD // 16):
            psl = pl.ds(off + t * 16, 16)
            ssrc_v[psl] = pad_src
            seid_v[psl] = pad_src
            sdl_v[psl] = pad_dl
        spill_chunk(spilled)
        ssl = pl.ds(SPILL, SPILL)
        hsl = pl.ds(base + (spilled + 1) * SPILL, SPILL)
        pltpu.sync_copy(ssrc_v.at[ssl], lsrc_hbm.at[hsl])
        pltpu.sync_copy(seid_v.at[ssl], leid_hbm.at[hsl])
        pltpu.sync_copy(sdl_v.at[ssl], ldl_hbm.at[hsl])

        total = spilled * SPILL + off
        cnt_v[pl.ds(0, 16)] = jnp.full((16,), 0, jnp.int32) + total
        pltpu.sync_copy(cnt_v, cnt_hbm.at[pl.ds(w * 16, 16)])

    return k(src, dst)


def _sc_edge(hw, ew, lsrc, leid, ldl, cnts):
    mesh = plsc.VectorSubcoreMesh(core_axis_name="c", subcore_axis_name="s")

    @functools.partial(
        pl.kernel,
        mesh=mesh,
        out_type=jax.ShapeDtypeStruct((M_PAD, H), jnp.float32),
        compiler_params=pltpu.CompilerParams(needs_layout_passes=False),
        scratch_types=[
            pltpu.VMEM((BE,), jnp.int32),
            pltpu.VMEM((BE,), jnp.int32),
            pltpu.VMEM((BE,), jnp.int32),
            pltpu.VMEM((BE,), jnp.int32),
            pltpu.VMEM((BE,), jnp.int32),
            pltpu.VMEM((BE,), jnp.int32),
            pltpu.VMEM((16,), jnp.int32),
            pltpu.VMEM((BE, H), jnp.float32),
            pltpu.VMEM((BE, H), jnp.float32),
            pltpu.VMEM((BE, H), jnp.float32),
            pltpu.VMEM((BE, H), jnp.float32),
            pltpu.VMEM((RPT + 1, H), jnp.float32),
            pltpu.SemaphoreType.DMA,
            pltpu.SemaphoreType.DMA,
            pltpu.SemaphoreType.DMA,
            pltpu.SemaphoreType.DMA,
            pltpu.SemaphoreType.DMA,
            pltpu.SemaphoreType.DMA,
        ],
    )
    def k(hw_hbm, ew_hbm, lsrc_hbm, leid_hbm, ldl_hbm, cnt_hbm, out_hbm,
          isrc0, isrc1, ieid0, ieid1, idl0, idl1, cnt_v,
          hwv0, hwv1, ewv0, ewv1, agg_v,
          semi0, semi1, semgh0, semgh1, semge0, semge1):
        c = lax.axis_index("c")
        s = lax.axis_index("s")
        w = s * 2 + c
        lo = w * RPT
        base = w * CAP
        isrc = (isrc0, isrc1)
        ieid = (ieid0, ieid1)
        idl = (idl0, idl1)
        hwv = (hwv0, hwv1)
        ewv = (ewv0, ewv1)
        semi = (semi0, semi1)
        semgh = (semgh0, semgh1)
        semge = (semge0, semge1)
        zeros16 = jnp.zeros((16,), jnp.float32)

        def zrow(r, cc):
            for j in range(H // 16):
                agg_v[r, pl.ds(j * 16, 16)] = zeros16
            return cc
        lax.fori_loop(0, RPT + 1, zrow, 0)

        pltpu.sync_copy(cnt_hbm.at[pl.ds(w * 16, 16)], cnt_v)
        cnt = cnt_v[pl.ds(0, 16)][0]
        nbp = lax.div(cnt + (2 * BE - 1), jnp.int32(2 * BE))

        def fire_idx(h, b):
            bsl = pl.ds(base + b * BE, BE)
            pltpu.async_copy(lsrc_hbm.at[bsl], isrc[h], semi[h])
            pltpu.async_copy(leid_hbm.at[bsl], ieid[h], semi[h])
            pltpu.async_copy(ldl_hbm.at[bsl], idl[h], semi[h])

        def wait_idx(h, b):
            bsl = pl.ds(base + b * BE, BE)
            pltpu.make_async_copy(lsrc_hbm.at[bsl], isrc[h], semi[h]).wait()
            pltpu.make_async_copy(leid_hbm.at[bsl], ieid[h], semi[h]).wait()
            pltpu.make_async_copy(ldl_hbm.at[bsl], idl[h], semi[h]).wait()

        def fire_g(h):
            pltpu.async_copy(hw_hbm.at[isrc[h]], hwv[h], semgh[h])
            pltpu.async_copy(ew_hbm.at[ieid[h]], ewv[h], semge[h])

        def wait_g(h):
            pltpu.make_async_copy(hw_hbm.at[isrc[h]], hwv[h], semgh[h]).wait()
            pltpu.make_async_copy(ew_hbm.at[ieid[h]], ewv[h], semge[h]).wait()

        def accum(h):
            def group(g, cc):
                dl = idl[h][pl.ds(g * 16, 16)]
                for lane in range(16):
                    e = g * 16 + lane
                    d = dl[lane]
                    for j in range(H // 16):
                        sl = pl.ds(j * 16, 16)
                        v = jnp.maximum(hwv[h][e, sl] + ewv[h][e, sl], 0.0)
                        plsc.addupdate(agg_v.at[d, sl], v)
                return cc
            lax.fori_loop(0, BE // 16, group, 0)

        fire_idx(0, jnp.int32(0))
        wait_idx(0, jnp.int32(0))
        fire_g(0)
        fire_idx(1, jnp.int32(1))

        def pair(p, cc):
            b0 = 2 * p
            wait_g(0)
            wait_idx(1, b0 + 1)
            fire_g(1)
            accum(0)
            fire_idx(0, b0 + 2)
            wait_g(1)
            wait_idx(0, b0 + 2)
            fire_g(0)
            accum(1)
            fire_idx(1, b0 + 3)
            return cc
        lax.fori_loop(0, nbp, pair, 0)

        wait_g(0)
        wait_idx(1, 2 * nbp + 1)

        pltpu.sync_copy(agg_v.at[pl.ds(0, RPT)], out_hbm.at[pl.ds(lo, RPT)])

    return k(hw, ew, lsrc, leid, ldl, cnts)


def kernel(x, edge_index, edge_attr, W_in, b_in,
           Wm0, We0, Ws0, Wa0, b0,
           Wm1, We1, Ws1, Wa1, b1,
           Wm2, We2, Ws2, Wa2, b2,
           W_out, b_out):
    src = edge_index[0]
    dst = edge_index[1]
    xp = jnp.zeros((M_PAD, D_IN), jnp.float32).at[:N].set(x)

    h, hw = _mm_in(xp, W_in, b_in.reshape(1, H), Wm0)
    ew0 = _mm_ew(edge_attr, We0)
    ew1 = _mm_ew(edge_attr, We1)
    ew2 = _mm_ew(edge_attr, We2)

    lsrc, leid, ldl, cnts = _sc_prepass(src, dst)

    agg = _sc_edge(hw, ew0, lsrc, leid, ldl, cnts)
    h, hw = _mm_update(h, agg, Ws0, Wa0, b0.reshape(1, H), Wm1)
    agg = _sc_edge(hw, ew1, lsrc, leid, ldl, cnts)
    h, hw = _mm_update(h, agg, Ws1, Wa1, b1.reshape(1, H), Wm2)
    agg = _sc_edge(hw, ew2, lsrc, leid, ldl, cnts)

    out = _mm_final(h, agg, Ws2, Wa2, b2.reshape(1, H),
                    W_out.reshape(1, H), b_out.reshape(1, 1))
    return out.reshape(-1)[:N]


# speedup vs baseline: 1.2804x; 1.2804x over previous
import functools

import jax
import jax.numpy as jnp
from jax import lax
from jax.experimental import pallas as pl
from jax.experimental.pallas import tpu as pltpu
from jax.experimental.pallas import tpu_sc as plsc

N = 10000
E = 320000
D_IN = 128
H = 256

M_PAD = 10240
BLK_M = 2048
BLK_E = 8000
HALF = M_PAD // 2
TRASH = HALF
AGG_ROWS = HALF + 8
NSUB = 16
E_PER_TILE = E // NSUB
BE = 80
NB = E_PER_TILE // BE
ROWS_PER_TILE = HALF // NSUB


def _mm_in(x, W_in, b_in, Wm0):
    def body(x_ref, w_ref, b_ref, wm_ref, h_ref, hw_ref):
        h = jnp.maximum(
            jnp.dot(x_ref[...], w_ref[...], preferred_element_type=jnp.float32)
            + b_ref[...], 0.0)
        h_ref[...] = h
        hw_ref[...] = jnp.dot(h, wm_ref[...], preferred_element_type=jnp.float32)

    return pl.pallas_call(
        body,
        grid=(M_PAD // BLK_M,),
        in_specs=[
            pl.BlockSpec((BLK_M, D_IN), lambda i: (i, 0)),
            pl.BlockSpec((D_IN, H), lambda i: (0, 0)),
            pl.BlockSpec((1, H), lambda i: (0, 0)),
            pl.BlockSpec((H, H), lambda i: (0, 0)),
        ],
        out_specs=[
            pl.BlockSpec((BLK_M, H), lambda i: (i, 0)),
            pl.BlockSpec((BLK_M, H), lambda i: (i, 0)),
        ],
        out_shape=[
            jax.ShapeDtypeStruct((M_PAD, H), jnp.float32),
            jax.ShapeDtypeStruct((M_PAD, H), jnp.float32),
        ],
    )(x, W_in, b_in, Wm0)


def _mm_ew(edge_attr, We):
    def body(a_ref, w_ref, o_ref):
        o_ref[...] = jnp.dot(a_ref[...], w_ref[...],
                             preferred_element_type=jnp.float32)

    return pl.pallas_call(
        body,
        grid=(E // BLK_E,),
        in_specs=[
            pl.BlockSpec((BLK_E, 16), lambda i: (i, 0)),
            pl.BlockSpec((16, H), lambda i: (0, 0)),
        ],
        out_specs=pl.BlockSpec((BLK_E, H), lambda i: (i, 0)),
        out_shape=jax.ShapeDtypeStruct((E, H), jnp.float32),
    )(edge_attr, We)


def _mm_update(h, agg, Ws, Wa, b, Wm_next):
    def body(h_ref, g_ref, ws_ref, wa_ref, b_ref, wm_ref, hn_ref, hw_ref):
        hn = jnp.maximum(
            jnp.dot(h_ref[...], ws_ref[...], preferred_element_type=jnp.float32)
            + jnp.dot(g_ref[...], wa_ref[...], preferred_element_type=jnp.float32)
            + b_ref[...], 0.0)
        hn_ref[...] = hn
        hw_ref[...] = jnp.dot(hn, wm_ref[...], preferred_element_type=jnp.float32)

    return pl.pallas_call(
        body,
        grid=(M_PAD // BLK_M,),
        in_specs=[
            pl.BlockSpec((BLK_M, H), lambda i: (i, 0)),
            pl.BlockSpec((BLK_M, H), lambda i: (i, 0)),
            pl.BlockSpec((H, H), lambda i: (0, 0)),
            pl.BlockSpec((H, H), lambda i: (0, 0)),
            pl.BlockSpec((1, H), lambda i: (0, 0)),
            pl.BlockSpec((H, H), lambda i: (0, 0)),
        ],
        out_specs=[
            pl.BlockSpec((BLK_M, H), lambda i: (i, 0)),
            pl.BlockSpec((BLK_M, H), lambda i: (i, 0)),
        ],
        out_shape=[
            jax.ShapeDtypeStruct((M_PAD, H), jnp.float32),
            jax.ShapeDtypeStruct((M_PAD, H), jnp.float32),
        ],
    )(h, agg, Ws, Wa, b, Wm_next)


def _mm_final(h, agg, Ws, Wa, b, w_out_row, b_out):
    def body(h_ref, g_ref, ws_ref, wa_ref, b_ref, wo_ref, bo_ref, o_ref):
        hn = jnp.maximum(
            jnp.dot(h_ref[...], ws_ref[...], preferred_element_type=jnp.float32)
            + jnp.dot(g_ref[...], wa_ref[...], preferred_element_type=jnp.float32)
            + b_ref[...], 0.0)
        o_ref[...] = jnp.sum(hn * wo_ref[...], axis=1, keepdims=True) + bo_ref[...]

    return pl.pallas_call(
        body,
        grid=(M_PAD // BLK_M,),
        in_specs=[
            pl.BlockSpec((BLK_M, H), lambda i: (i, 0)),
            pl.BlockSpec((BLK_M, H), lambda i: (i, 0)),
            pl.BlockSpec((H, H), lambda i: (0, 0)),
            pl.BlockSpec((H, H), lambda i: (0, 0)),
            pl.BlockSpec((1, H), lambda i: (0, 0)),
            pl.BlockSpec((1, H), lambda i: (0, 0)),
            pl.BlockSpec((1, 1), lambda i: (0, 0)),
        ],
        out_specs=pl.BlockSpec((BLK_M, 1), lambda i: (i, 0)),
        out_shape=jax.ShapeDtypeStruct((M_PAD, 1), jnp.float32),
    )(h, agg, Ws, Wa, b, w_out_row, b_out)


NW = 32
RPT = M_PAD // NW
BD = 2000
NBLK = E // BD
SPILL = 1024
STAGE = 2 * SPILL + 16
CAP = 312 * SPILL + 2 * SPILL
PAD = 112


def _sc_prepass(src, dst):
    mesh = plsc.VectorSubcoreMesh(core_axis_name="c", subcore_axis_name="s")

    @functools.partial(
        pl.kernel,
        mesh=mesh,
        out_type=[
            jax.ShapeDtypeStruct((NW * CAP,), jnp.int32),
            jax.ShapeDtypeStruct((NW * CAP,), jnp.int32),
            jax.ShapeDtypeStruct((NW * CAP,), jnp.int32),
            jax.ShapeDtypeStruct((NW * 16,), jnp.int32),
        ],
        compiler_params=pltpu.CompilerParams(needs_layout_passes=False),
        scratch_types=[
            pltpu.VMEM((BD,), jnp.int32),
            pltpu.VMEM((BD,), jnp.int32),
            pltpu.VMEM((BD,), jnp.int32),
            pltpu.VMEM((BD,), jnp.int32),
            pltpu.VMEM((STAGE,), jnp.int32),
            pltpu.VMEM((STAGE,), jnp.int32),
            pltpu.VMEM((STAGE,), jnp.int32),
            pltpu.VMEM((16,), jnp.int32),
            pltpu.SemaphoreType.DMA,
            pltpu.SemaphoreType.DMA,
        ],
    )
    def k(src_hbm, dst_hbm, lsrc_hbm, leid_hbm, ldl_hbm, cnt_hbm,
          sblk0, sblk1, dblk0, dblk1, ssrc_v, seid_v, sdl_v, cnt_v,
          semb0, semb1):
        c = lax.axis_index("c")
        s = lax.axis_index("s")
        w = s * 2 + c
        lo = w * RPT
        base = w * CAP
        sblk = (sblk0, sblk1)
        dblk = (dblk0, dblk1)
        semb = (semb0, semb1)

        def blk_off(blk):
            return jnp.minimum(blk, NBLK - 1) * BD

        def fire_blk(h, blk):
            o = blk_off(blk)
            pltpu.async_copy(src_hbm.at[pl.ds(o, BD)], sblk[h], semb[h])
            pltpu.async_copy(dst_hbm.at[pl.ds(o, BD)], dblk[h], semb[h])

        def wait_blk(h, blk):
            o = blk_off(blk)
            pltpu.make_async_copy(src_hbm.at[pl.ds(o, BD)], sblk[h], semb[h]).wait()
            pltpu.make_async_copy(dst_hbm.at[pl.ds(o, BD)], dblk[h], semb[h]).wait()

        def spill_chunk(spilled):
            hsl = pl.ds(base + spilled * SPILL, SPILL)
            vsl = pl.ds(0, SPILL)
            pltpu.sync_copy(ssrc_v.at[vsl], lsrc_hbm.at[hsl])
            pltpu.sync_copy(seid_v.at[vsl], leid_hbm.at[hsl])
            pltpu.sync_copy(sdl_v.at[vsl], ldl_hbm.at[hsl])

        def scan_half(h, blk, carry):
            def chunk_body(i, carry):
                off, spilled = carry
                sl = pl.ds(i * 16, 16)
                dv = dblk[h][sl]
                sv = sblk[h][sl]
                dloc = dv - lo
                own = (dloc >= 0) & (dloc < RPT)
                eid = blk * BD + i * 16 + lax.iota(jnp.int32, 16)
                cnt = plsc.all_reduce_population_count(own)[0]

                def do_stage(off):
                    osl = pl.ds(off, 16)
                    plsc.store_compressed(ssrc_v.at[osl], sv, mask=own)
                    plsc.store_compressed(seid_v.at[osl], eid, mask=own)
                    plsc.store_compressed(sdl_v.at[osl], dloc, mask=own)
                    return off + cnt

                off = lax.cond(cnt > 0, do_stage, lambda off: off, off)

                def spill(args):
                    off, spilled = args
                    spill_chunk(spilled)
                    rsl = pl.ds(SPILL, 16)
                    rs, re, rd = ssrc_v[rsl], seid_v[rsl], sdl_v[rsl]
                    tsl = pl.ds(0, 16)
                    ssrc_v[tsl] = rs
                    seid_v[tsl] = re
                    sdl_v[tsl] = rd
                    return off - SPILL, spilled + 1

                return lax.cond(off >= SPILL, spill, lambda a: a, (off, spilled))
            return lax.fori_loop(0, BD // 16, chunk_body, carry)

        fire_blk(0, 0)

        def pair(p, carry):
            b0 = 2 * p
            wait_blk(0, b0)
            fire_blk(1, b0 + 1)
            carry = scan_half(0, b0, carry)
            wait_blk(1, b0 + 1)
            fire_blk(0, b0 + 2)
            carry = scan_half(1, b0 + 1, carry)
            return carry

        off, spilled = lax.fori_loop(0, NBLK // 2, pair,
                                     (jnp.int32(0), jnp.int32(0)))
        wait_blk(0, NBLK)

        pad_src = jnp.zeros((16,), jnp.int32)
        pad_dl = jnp.full((16,), RPT, jnp.int32)
        for t in range(PAD // 16):
            psl = pl.ds(off + t * 16, 16)
            ssrc_v[psl] = pad_src
            seid_v[psl] = pad_src
            sdl_v[psl] = pad_dl
        spill_chunk(spilled)
        ssl = pl.ds(SPILL, SPILL)
        hsl = pl.ds(base + (spilled + 1) * SPILL, SPILL)
        pltpu.sync_copy(ssrc_v.at[ssl], lsrc_hbm.at[hsl])
        pltpu.sync_copy(seid_v.at[ssl], leid_hbm.at[hsl])
        pltpu.sync_copy(sdl_v.at[ssl], ldl_hbm.at[hsl])

        total = spilled * SPILL + off
        cnt_v[pl.ds(0, 16)] = jnp.full((16,), 0, jnp.int32) + total
        pltpu.sync_copy(cnt_v, cnt_hbm.at[pl.ds(w * 16, 16)])

    return k(src, dst)


IBB = 12
IB = IBB * BE


def _sc_edge(hw, ew, lsrc, leid, ldl, cnts):
    mesh = plsc.VectorSubcoreMesh(core_axis_name="c", subcore_axis_name="s")

    @functools.partial(
        pl.kernel,
        mesh=mesh,
        out_type=jax.ShapeDtypeStruct((M_PAD, H), jnp.float32),
        compiler_params=pltpu.CompilerParams(needs_layout_passes=False),
        scratch_types=[
            pltpu.VMEM((IB,), jnp.int32),
            pltpu.VMEM((IB,), jnp.int32),
            pltpu.VMEM((IB,), jnp.int32),
            pltpu.VMEM((16,), jnp.int32),
            pltpu.VMEM((BE, H), jnp.float32),
            pltpu.VMEM((BE, H), jnp.float32),
            pltpu.VMEM((RPT + 1, H), jnp.float32),
            pltpu.SemaphoreType.DMA,
            pltpu.SemaphoreType.DMA,
        ],
    )
    def k(hw_hbm, ew_hbm, lsrc_hbm, leid_hbm, ldl_hbm, cnt_hbm, out_hbm,
          isrc_v, ieid_v, idl_v, cnt_v, hw_v, ew_v, agg_v, sem1, sem2):
        c = lax.axis_index("c")
        s = lax.axis_index("s")
        w = s * 2 + c
        lo = w * RPT
        base = w * CAP
        zeros16 = jnp.zeros((16,), jnp.float32)

        def zrow(r, cc):
            for j in range(H // 16):
                agg_v[r, pl.ds(j * 16, 16)] = zeros16
            return cc
        lax.fori_loop(0, RPT + 1, zrow, 0)

        pltpu.sync_copy(cnt_hbm.at[pl.ds(w * 16, 16)], cnt_v)
        cnt = cnt_v[pl.ds(0, 16)][0]
        nb = lax.div(cnt + (BE - 1), jnp.int32(BE))
        nblk = lax.div(nb + (IBB - 1), jnp.int32(IBB))

        def blk_body(bk, cc):
            bsl = pl.ds(base + bk * IB, IB)
            cp1 = pltpu.async_copy(lsrc_hbm.at[bsl], isrc_v, sem1)
            cp2 = pltpu.async_copy(leid_hbm.at[bsl], ieid_v, sem2)
            cp3 = pltpu.async_copy(ldl_hbm.at[bsl], idl_v, sem1)
            cp1.wait()
            cp2.wait()
            cp3.wait()
            nbb = jnp.minimum(nb - bk * IBB, IBB)

            def batch(bb, cc2):
                osl = pl.ds(bb * BE, BE)
                g1 = pltpu.async_copy(hw_hbm.at[isrc_v.at[osl]], hw_v, sem1)
                g2 = pltpu.async_copy(ew_hbm.at[ieid_v.at[osl]], ew_v, sem2)
                g1.wait()
                g2.wait()

                def group(g, cc3):
                    dl = idl_v[pl.ds(bb * BE + g * 16, 16)]
                    for lane in range(16):
                        e = g * 16 + lane
                        d = dl[lane]
                        for j in range(H // 16):
                            sl = pl.ds(j * 16, 16)
                            v = jnp.maximum(hw_v[e, sl] + ew_v[e, sl], 0.0)
                            plsc.addupdate(agg_v.at[d, sl], v)
                    return cc3
                lax.fori_loop(0, BE // 16, group, 0)
                return cc2
            lax.fori_loop(0, nbb, batch, 0)
            return cc
        lax.fori_loop(0, nblk, blk_body, 0)

        pltpu.sync_copy(agg_v.at[pl.ds(0, RPT)], out_hbm.at[pl.ds(lo, RPT)])

    return k(hw, ew, lsrc, leid, ldl, cnts)


def kernel(x, edge_index, edge_attr, W_in, b_in,
           Wm0, We0, Ws0, Wa0, b0,
           Wm1, We1, Ws1, Wa1, b1,
           Wm2, We2, Ws2, Wa2, b2,
           W_out, b_out):
    src = edge_index[0]
    dst = edge_index[1]
    xp = jnp.zeros((M_PAD, D_IN), jnp.float32).at[:N].set(x)

    h, hw = _mm_in(xp, W_in, b_in.reshape(1, H), Wm0)
    ew0 = _mm_ew(edge_attr, We0)
    ew1 = _mm_ew(edge_attr, We1)
    ew2 = _mm_ew(edge_attr, We2)

    lsrc, leid, ldl, cnts = _sc_prepass(src, dst)

    agg = _sc_edge(hw, ew0, lsrc, leid, ldl, cnts)
    h, hw = _mm_update(h, agg, Ws0, Wa0, b0.reshape(1, H), Wm1)
    agg = _sc_edge(hw, ew1, lsrc, leid, ldl, cnts)
    h, hw = _mm_update(h, agg, Ws1, Wa1, b1.reshape(1, H), Wm2)
    agg = _sc_edge(hw, ew2, lsrc, leid, ldl, cnts)

    out = _mm_final(h, agg, Ws2, Wa2, b2.reshape(1, H),
                    W_out.reshape(1, H), b_out.reshape(1, 1))
    return out.reshape(-1)[:N]


# speedup vs baseline: 1.9633x; 1.5334x over previous
import functools

import jax
import jax.numpy as jnp
from jax import lax
from jax.experimental import pallas as pl
from jax.experimental.pallas import tpu as pltpu
from jax.experimental.pallas import tpu_sc as plsc

N = 10000
E = 320000
D_IN = 128
H = 256

M_PAD = 10240
BLK_M = 2048
BLK_E = 8000
HALF = M_PAD // 2
TRASH = HALF
AGG_ROWS = HALF + 8
NSUB = 16
E_PER_TILE = E // NSUB
BE = 80
NB = E_PER_TILE // BE
ROWS_PER_TILE = HALF // NSUB


def _pack_bf16(y):
    yb = y.astype(jnp.bfloat16).astype(jnp.float32)
    u = jax.lax.bitcast_convert_type(yb, jnp.uint32)
    lo = u[:, :H // 2] >> 16
    hi = u[:, H // 2:] & jnp.uint32(0xFFFF0000)
    return jax.lax.bitcast_convert_type(lo | hi, jnp.int32)


def _mm_in(x, W_in, b_in, Wm0):
    def body(x_ref, w_ref, b_ref, wm_ref, h_ref, hw_ref):
        h = jnp.maximum(
            jnp.dot(x_ref[...], w_ref[...], preferred_element_type=jnp.float32)
            + b_ref[...], 0.0)
        h_ref[...] = h
        hw_ref[...] = _pack_bf16(
            jnp.dot(h, wm_ref[...], preferred_element_type=jnp.float32))

    return pl.pallas_call(
        body,
        grid=(M_PAD // BLK_M,),
        in_specs=[
            pl.BlockSpec((BLK_M, D_IN), lambda i: (i, 0)),
            pl.BlockSpec((D_IN, H), lambda i: (0, 0)),
            pl.BlockSpec((1, H), lambda i: (0, 0)),
            pl.BlockSpec((H, H), lambda i: (0, 0)),
        ],
        out_specs=[
            pl.BlockSpec((BLK_M, H), lambda i: (i, 0)),
            pl.BlockSpec((BLK_M, H // 2), lambda i: (i, 0)),
        ],
        out_shape=[
            jax.ShapeDtypeStruct((M_PAD, H), jnp.float32),
            jax.ShapeDtypeStruct((M_PAD, H // 2), jnp.int32),
        ],
    )(x, W_in, b_in, Wm0)


def _mm_ew(edge_attr, We):
    def body(a_ref, w_ref, o_ref):
        o_ref[...] = _pack_bf16(jnp.dot(a_ref[...], w_ref[...],
                                        preferred_element_type=jnp.float32))

    return pl.pallas_call(
        body,
        grid=(E // BLK_E,),
        in_specs=[
            pl.BlockSpec((BLK_E, 16), lambda i: (i, 0)),
            pl.BlockSpec((16, H), lambda i: (0, 0)),
        ],
        out_specs=pl.BlockSpec((BLK_E, H // 2), lambda i: (i, 0)),
        out_shape=jax.ShapeDtypeStruct((E, H // 2), jnp.int32),
    )(edge_attr, We)


def _mm_update(h, agg, Ws, Wa, b, Wm_next):
    def body(h_ref, g_ref, ws_ref, wa_ref, b_ref, wm_ref, hn_ref, hw_ref):
        hn = jnp.maximum(
            jnp.dot(h_ref[...], ws_ref[...], preferred_element_type=jnp.float32)
            + jnp.dot(g_ref[...], wa_ref[...], preferred_element_type=jnp.float32)
            + b_ref[...], 0.0)
        hn_ref[...] = hn
        hw_ref[...] = _pack_bf16(
            jnp.dot(hn, wm_ref[...], preferred_element_type=jnp.float32))

    return pl.pallas_call(
        body,
        grid=(M_PAD // BLK_M,),
        in_specs=[
            pl.BlockSpec((BLK_M, H), lambda i: (i, 0)),
            pl.BlockSpec((BLK_M, H), lambda i: (i, 0)),
            pl.BlockSpec((H, H), lambda i: (0, 0)),
            pl.BlockSpec((H, H), lambda i: (0, 0)),
            pl.BlockSpec((1, H), lambda i: (0, 0)),
            pl.BlockSpec((H, H), lambda i: (0, 0)),
        ],
        out_specs=[
            pl.BlockSpec((BLK_M, H), lambda i: (i, 0)),
            pl.BlockSpec((BLK_M, H // 2), lambda i: (i, 0)),
        ],
        out_shape=[
            jax.ShapeDtypeStruct((M_PAD, H), jnp.float32),
            jax.ShapeDtypeStruct((M_PAD, H // 2), jnp.int32),
        ],
    )(h, agg, Ws, Wa, b, Wm_next)


def _mm_final(h, agg, Ws, Wa, b, w_out_row, b_out):
    def body(h_ref, g_ref, ws_ref, wa_ref, b_ref, wo_ref, bo_ref, o_ref):
        hn = jnp.maximum(
            jnp.dot(h_ref[...], ws_ref[...], preferred_element_type=jnp.float32)
            + jnp.dot(g_ref[...], wa_ref[...], preferred_element_type=jnp.float32)
            + b_ref[...], 0.0)
        o_ref[...] = jnp.sum(hn * wo_ref[...], axis=1, keepdims=True) + bo_ref[...]

    return pl.pallas_call(
        body,
        grid=(M_PAD // BLK_M,),
        in_specs=[
            pl.BlockSpec((BLK_M, H), lambda i: (i, 0)),
            pl.BlockSpec((BLK_M, H), lambda i: (i, 0)),
            pl.BlockSpec((H, H), lambda i: (0, 0)),
            pl.BlockSpec((H, H), lambda i: (0, 0)),
            pl.BlockSpec((1, H), lambda i: (0, 0)),
            pl.BlockSpec((1, H), lambda i: (0, 0)),
            pl.BlockSpec((1, 1), lambda i: (0, 0)),
        ],
        out_specs=pl.BlockSpec((BLK_M, 1), lambda i: (i, 0)),
        out_shape=jax.ShapeDtypeStruct((M_PAD, 1), jnp.float32),
    )(h, agg, Ws, Wa, b, w_out_row, b_out)


NW = 32
RPT = M_PAD // NW
BD = 2000
NBLK = E // BD
SPILL = 1024
STAGE = 2 * SPILL + 16
CAP = 312 * SPILL + 2 * SPILL
PAD = 112


def _sc_prepass(src, dst):
    mesh = plsc.VectorSubcoreMesh(core_axis_name="c", subcore_axis_name="s")

    @functools.partial(
        pl.kernel,
        mesh=mesh,
        out_type=[
            jax.ShapeDtypeStruct((NW * CAP,), jnp.int32),
            jax.ShapeDtypeStruct((NW * CAP,), jnp.int32),
            jax.ShapeDtypeStruct((NW * CAP,), jnp.int32),
            jax.ShapeDtypeStruct((NW * 16,), jnp.int32),
        ],
        compiler_params=pltpu.CompilerParams(needs_layout_passes=False),
        scratch_types=[
            pltpu.VMEM((BD,), jnp.int32),
            pltpu.VMEM((BD,), jnp.int32),
            pltpu.VMEM((BD,), jnp.int32),
            pltpu.VMEM((BD,), jnp.int32),
            pltpu.VMEM((STAGE,), jnp.int32),
            pltpu.VMEM((STAGE,), jnp.int32),
            pltpu.VMEM((STAGE,), jnp.int32),
            pltpu.VMEM((16,), jnp.int32),
            pltpu.SemaphoreType.DMA,
            pltpu.SemaphoreType.DMA,
        ],
    )
    def k(src_hbm, dst_hbm, lsrc_hbm, leid_hbm, ldl_hbm, cnt_hbm,
          sblk0, sblk1, dblk0, dblk1, ssrc_v, seid_v, sdl_v, cnt_v,
          semb0, semb1):
        c = lax.axis_index("c")
        s = lax.axis_index("s")
        w = s * 2 + c
        lo = w * RPT
        base = w * CAP
        sblk = (sblk0, sblk1)
        dblk = (dblk0, dblk1)
        semb = (semb0, semb1)

        def blk_off(blk):
            return jnp.minimum(blk, NBLK - 1) * BD

        def fire_blk(h, blk):
            o = blk_off(blk)
            pltpu.async_copy(src_hbm.at[pl.ds(o, BD)], sblk[h], semb[h])
            pltpu.async_copy(dst_hbm.at[pl.ds(o, BD)], dblk[h], semb[h])

        def wait_blk(h, blk):
            o = blk_off(blk)
            pltpu.make_async_copy(src_hbm.at[pl.ds(o, BD)], sblk[h], semb[h]).wait()
            pltpu.make_async_copy(dst_hbm.at[pl.ds(o, BD)], dblk[h], semb[h]).wait()

        def spill_chunk(spilled):
            hsl = pl.ds(base + spilled * SPILL, SPILL)
            vsl = pl.ds(0, SPILL)
            pltpu.sync_copy(ssrc_v.at[vsl], lsrc_hbm.at[hsl])
            pltpu.sync_copy(seid_v.at[vsl], leid_hbm.at[hsl])
            pltpu.sync_copy(sdl_v.at[vsl], ldl_hbm.at[hsl])

        def scan_half(h, blk, carry):
            def chunk_body(i, carry):
                off, spilled = carry
                sl = pl.ds(i * 16, 16)
                dv = dblk[h][sl]
                sv = sblk[h][sl]
                dloc = dv - lo
                own = (dloc >= 0) & (dloc < RPT)
                eid = blk * BD + i * 16 + lax.iota(jnp.int32, 16)
                cnt = plsc.all_reduce_population_count(own)[0]

                def do_stage(off):
                    osl = pl.ds(off, 16)
                    plsc.store_compressed(ssrc_v.at[osl], sv, mask=own)
                    plsc.store_compressed(seid_v.at[osl], eid, mask=own)
                    plsc.store_compressed(sdl_v.at[osl], dloc, mask=own)
                    return off + cnt

                off = lax.cond(cnt > 0, do_stage, lambda off: off, off)

                def spill(args):
                    off, spilled = args
                    spill_chunk(spilled)
                    rsl = pl.ds(SPILL, 16)
                    rs, re, rd = ssrc_v[rsl], seid_v[rsl], sdl_v[rsl]
                    tsl = pl.ds(0, 16)
                    ssrc_v[tsl] = rs
                    seid_v[tsl] = re
                    sdl_v[tsl] = rd
                    return off - SPILL, spilled + 1

                return lax.cond(off >= SPILL, spill, lambda a: a, (off, spilled))
            return lax.fori_loop(0, BD // 16, chunk_body, carry)

        fire_blk(0, 0)

        def pair(p, carry):
            b0 = 2 * p
            wait_blk(0, b0)
            fire_blk(1, b0 + 1)
            carry = scan_half(0, b0, carry)
            wait_blk(1, b0 + 1)
            fire_blk(0, b0 + 2)
            carry = scan_half(1, b0 + 1, carry)
            return carry

        off, spilled = lax.fori_loop(0, NBLK // 2, pair,
                                     (jnp.int32(0), jnp.int32(0)))
        wait_blk(0, NBLK)

        pad_src = jnp.zeros((16,), jnp.int32)
        pad_dl = jnp.full((16,), RPT, jnp.int32)
        for t in range(PAD // 16):
            psl = pl.ds(off + t * 16, 16)
            ssrc_v[psl] = pad_src
            seid_v[psl] = pad_src
            sdl_v[psl] = pad_dl
        spill_chunk(spilled)
        ssl = pl.ds(SPILL, SPILL)
        hsl = pl.ds(base + (spilled + 1) * SPILL, SPILL)
        pltpu.sync_copy(ssrc_v.at[ssl], lsrc_hbm.at[hsl])
        pltpu.sync_copy(seid_v.at[ssl], leid_hbm.at[hsl])
        pltpu.sync_copy(sdl_v.at[ssl], ldl_hbm.at[hsl])

        total = spilled * SPILL + off
        cnt_v[pl.ds(0, 16)] = jnp.full((16,), 0, jnp.int32) + total
        pltpu.sync_copy(cnt_v, cnt_hbm.at[pl.ds(w * 16, 16)])

    return k(src, dst)


IBB = 12
IB = IBB * BE


def _sc_edge(hw, ew, lsrc, leid, ldl, cnts):
    mesh = plsc.VectorSubcoreMesh(core_axis_name="c", subcore_axis_name="s")

    @functools.partial(
        pl.kernel,
        mesh=mesh,
        out_type=jax.ShapeDtypeStruct((M_PAD, H), jnp.float32),
        compiler_params=pltpu.CompilerParams(needs_layout_passes=False),
        scratch_types=[
            pltpu.VMEM((IB,), jnp.int32),
            pltpu.VMEM((IB,), jnp.int32),
            pltpu.VMEM((IB,), jnp.int32),
            pltpu.VMEM((16,), jnp.int32),
            pltpu.VMEM((BE, H // 2), jnp.int32),
            pltpu.VMEM((BE, H // 2), jnp.int32),
            pltpu.VMEM((RPT + 1, H), jnp.float32),
            pltpu.SemaphoreType.DMA,
            pltpu.SemaphoreType.DMA,
        ],
    )
    def k(hw_hbm, ew_hbm, lsrc_hbm, leid_hbm, ldl_hbm, cnt_hbm, out_hbm,
          isrc_v, ieid_v, idl_v, cnt_v, hw_v, ew_v, agg_v, sem1, sem2):
        c = lax.axis_index("c")
        s = lax.axis_index("s")
        w = s * 2 + c
        lo = w * RPT
        base = w * CAP
        zeros16 = jnp.zeros((16,), jnp.float32)

        def zrow(r, cc):
            for j in range(H // 16):
                agg_v[r, pl.ds(j * 16, 16)] = zeros16
            return cc
        lax.fori_loop(0, RPT + 1, zrow, 0)

        pltpu.sync_copy(cnt_hbm.at[pl.ds(w * 16, 16)], cnt_v)
        cnt = cnt_v[pl.ds(0, 16)][0]
        nb = lax.div(cnt + (BE - 1), jnp.int32(BE))
        nblk = lax.div(nb + (IBB - 1), jnp.int32(IBB))

        def blk_body(bk, cc):
            bsl = pl.ds(base + bk * IB, IB)
            cp1 = pltpu.async_copy(lsrc_hbm.at[bsl], isrc_v, sem1)
            cp2 = pltpu.async_copy(leid_hbm.at[bsl], ieid_v, sem2)
            cp3 = pltpu.async_copy(ldl_hbm.at[bsl], idl_v, sem1)
            cp1.wait()
            cp2.wait()
            cp3.wait()
            nbb = jnp.minimum(nb - bk * IBB, IBB)

            def batch(bb, cc2):
                osl = pl.ds(bb * BE, BE)
                g1 = pltpu.async_copy(hw_hbm.at[isrc_v.at[osl]], hw_v, sem1)
                g2 = pltpu.async_copy(ew_hbm.at[ieid_v.at[osl]], ew_v, sem2)
                g1.wait()
                g2.wait()

                def group(g, cc3):
                    dl = idl_v[pl.ds(bb * BE + g * 16, 16)]
                    hi16 = jnp.full((16,), -65536, jnp.int32)
                    for lane in range(16):
                        e = g * 16 + lane
                        d = dl[lane]
                        for jp in range(H // 32):
                            psl = pl.ds(jp * 16, 16)
                            u1 = hw_v[e, psl]
                            u2 = ew_v[e, psl]
                            he = plsc.bitcast(u1 << 16, jnp.float32)
                            ho = plsc.bitcast(u1 & hi16, jnp.float32)
                            ee = plsc.bitcast(u2 << 16, jnp.float32)
                            eo = plsc.bitcast(u2 & hi16, jnp.float32)
                            ve = jnp.maximum(he + ee, 0.0)
                            vo = jnp.maximum(ho + eo, 0.0)
                            plsc.addupdate(agg_v.at[d, pl.ds(jp * 16, 16)], ve)
                            plsc.addupdate(
                                agg_v.at[d, pl.ds(H // 2 + jp * 16, 16)], vo)
                    return cc3
                lax.fori_loop(0, BE // 16, group, 0)
                return cc2
            lax.fori_loop(0, nbb, batch, 0)
            return cc
        lax.fori_loop(0, nblk, blk_body, 0)

        pltpu.sync_copy(agg_v.at[pl.ds(0, RPT)], out_hbm.at[pl.ds(lo, RPT)])

    return k(hw, ew, lsrc, leid, ldl, cnts)


def kernel(x, edge_index, edge_attr, W_in, b_in,
           Wm0, We0, Ws0, Wa0, b0,
           Wm1, We1, Ws1, Wa1, b1,
           Wm2, We2, Ws2, Wa2, b2,
           W_out, b_out):
    src = edge_index[0]
    dst = edge_index[1]
    xp = jnp.zeros((M_PAD, D_IN), jnp.float32).at[:N].set(x)

    h, hw = _mm_in(xp, W_in, b_in.reshape(1, H), Wm0)
    ew0 = _mm_ew(edge_attr, We0)
    ew1 = _mm_ew(edge_attr, We1)
    ew2 = _mm_ew(edge_attr, We2)

    lsrc, leid, ldl, cnts = _sc_prepass(src, dst)

    agg = _sc_edge(hw, ew0, lsrc, leid, ldl, cnts)
    h, hw = _mm_update(h, agg, Ws0, Wa0, b0.reshape(1, H), Wm1)
    agg = _sc_edge(hw, ew1, lsrc, leid, ldl, cnts)
    h, hw = _mm_update(h, agg, Ws1, Wa1, b1.reshape(1, H), Wm2)
    agg = _sc_edge(hw, ew2, lsrc, leid, ldl, cnts)

    out = _mm_final(h, agg, Ws2, Wa2, b2.reshape(1, H),
                    W_out.reshape(1, H), b_out.reshape(1, 1))
    return out.reshape(-1)[:N]
